# baseline (speedup 1.0000x reference)
import functools

import jax
import jax.numpy as jnp
from jax import lax
from jax.experimental import pallas as pl
from jax.experimental.pallas import tpu as pltpu
from jax.experimental.pallas import tpu_sc as plsc

N = 10000
E = 320000
D = 128
H = 32
G = 64

NCORES = 2
NSUB = 16
NW = NCORES * NSUB
CH = 128
NCHG = E // CH
NCHW = NCHG // NW
NEXTRA = NCHG - NCHW * NW
NPAD = 10240
RPT = NPAD // NSUB
DEGW = 16


def _deg_body(edge_hbm, out_hbm, dst_v, ones_v, zero_v, deg_sh, sem):
    cid = lax.axis_index("c")
    sid = lax.axis_index("s")
    wid = sid * NCORES + cid
    pltpu.sync_copy(edge_hbm.at[1, pl.ds(wid * NCHW, NCHW)],
                    dst_v.at[pl.ds(0, NCHW)])

    @pl.when(wid < NEXTRA)
    def _load_extra():
        pltpu.sync_copy(edge_hbm.at[1, NCHW * NW + wid], dst_v.at[NCHW])

    def _fill(r, carry):
        ones_v[r, :] = jnp.ones((DEGW,), jnp.float32)
        zero_v[r, :] = jnp.zeros((DEGW,), jnp.float32)
        return carry

    lax.fori_loop(0, CH, _fill, 0)
    for q in range(RPT // CH):
        pltpu.sync_copy(zero_v, deg_sh.at[pl.ds(sid * RPT + q * CH, CH)])
    plsc.subcore_barrier()

    def _scat(k, carry):
        for b in range(6):
            pltpu.async_copy(ones_v, deg_sh.at[dst_v.at[6 * k + b]], sem,
                             add=True)
        for b in range(6):
            pltpu.make_async_copy(ones_v, deg_sh.at[dst_v.at[6 * k + b]],
                                  sem).wait()
        return carry

    lax.fori_loop(0, NCHW // 6, _scat, 0)

    @pl.when(wid < NEXTRA)
    def _scat_extra():
        pltpu.sync_copy(ones_v, deg_sh.at[dst_v.at[NCHW]], add=True)

    plsc.subcore_barrier()
    pltpu.sync_copy(deg_sh.at[pl.ds(sid * RPT, RPT)],
                    out_hbm.at[pl.ds(cid * NPAD + sid * RPT, RPT)])


def _msg_body(h_hbm, edge_hbm, out_hbm,
              src_v, dst_v, buf0, buf1, h_sh, acc_sh, sem0, sem1):
    cid = lax.axis_index("c")
    sid = lax.axis_index("s")
    wid = sid * NCORES + cid
    pltpu.sync_copy(edge_hbm.at[0, pl.ds(wid * NCHW, NCHW)],
                    src_v.at[pl.ds(0, NCHW)])
    pltpu.sync_copy(edge_hbm.at[1, pl.ds(wid * NCHW, NCHW)],
                    dst_v.at[pl.ds(0, NCHW)])

    @pl.when(wid < NEXTRA)
    def _load_extra():
        pltpu.sync_copy(edge_hbm.at[0, NCHW * NW + wid], src_v.at[NCHW])
        pltpu.sync_copy(edge_hbm.at[1, NCHW * NW + wid], dst_v.at[NCHW])

    def _zfill(r, carry):
        buf0[r, pl.ds(0, 16)] = jnp.zeros((16,), jnp.float32)
        buf0[r, pl.ds(16, 16)] = jnp.zeros((16,), jnp.float32)
        return carry

    lax.fori_loop(0, CH, _zfill, 0)
    sl = pl.ds(sid * RPT, RPT)
    for q in range(RPT // CH):
        pltpu.sync_copy(buf0, acc_sh.at[pl.ds(sid * RPT + q * CH, CH)])
    hsl = pl.ds(sid * (N // NSUB), N // NSUB)
    pltpu.sync_copy(h_hbm.at[hsl], h_sh.at[hsl])
    plsc.subcore_barrier()
    pltpu.async_copy(h_sh.at[src_v.at[0]], buf0, sem0)

    def _pair(k, carry):
        j = 2 * k
        pltpu.make_async_copy(h_sh.at[src_v.at[j]], buf0, sem0).wait()
        pltpu.async_copy(h_sh.at[src_v.at[j + 1]], buf1, sem1)
        pltpu.sync_copy(buf0, acc_sh.at[dst_v.at[j]], add=True)
        pltpu.make_async_copy(h_sh.at[src_v.at[j + 1]], buf1, sem1).wait()

        @pl.when(k < NCHW // 2 - 1)
        def _prefetch():
            pltpu.async_copy(h_sh.at[src_v.at[j + 2]], buf0, sem0)

        pltpu.sync_copy(buf1, acc_sh.at[dst_v.at[j + 1]], add=True)
        return carry

    lax.fori_loop(0, NCHW // 2, _pair, 0)

    @pl.when(wid < NEXTRA)
    def _extra():
        pltpu.async_copy(h_sh.at[src_v.at[NCHW]], buf0, sem0).wait()
        pltpu.sync_copy(buf0, acc_sh.at[dst_v.at[NCHW]], add=True)

    plsc.subcore_barrier()
    pltpu.sync_copy(acc_sh.at[sl],
                    out_hbm.at[pl.ds(cid * NPAD + sid * RPT, RPT)])


@functools.cache
def _build_sc_kernels():
    mesh = plsc.VectorSubcoreMesh(core_axis_name="c", subcore_axis_name="s",
                                  num_cores=NCORES, num_subcores=NSUB)
    params = pltpu.CompilerParams(use_tc_tiling_on_sc=False)
    deg_kernel = pl.kernel(
        _deg_body,
        out_type=jax.ShapeDtypeStruct((NCORES * NPAD, DEGW), jnp.float32),
        mesh=mesh,
        compiler_params=params,
        scratch_types=[
            pltpu.VMEM((NCHW + 1, CH), jnp.int32),
            pltpu.VMEM((CH, DEGW), jnp.float32),
            pltpu.VMEM((CH, DEGW), jnp.float32),
            pltpu.VMEM_SHARED((NPAD, DEGW), jnp.float32),
            pltpu.SemaphoreType.DMA,
        ],
    )
    msg_kernel = pl.kernel(
        _msg_body,
        out_type=jax.ShapeDtypeStruct((NCORES * NPAD, H), jnp.float32),
        mesh=mesh,
        compiler_params=params,
        scratch_types=[
            pltpu.VMEM((NCHW + 1, CH), jnp.int32),
            pltpu.VMEM((NCHW + 1, CH), jnp.int32),
            pltpu.VMEM((CH, H), jnp.float32),
            pltpu.VMEM((CH, H), jnp.float32),
            pltpu.VMEM_SHARED((N, H), jnp.float32),
            pltpu.VMEM_SHARED((NPAD, H), jnp.float32),
            pltpu.SemaphoreType.DMA,
            pltpu.SemaphoreType.DMA,
        ],
    )
    return deg_kernel, msg_kernel


NB = 10
BR = N // NB


def _prep_body(x_ref, w1_ref, dg0_ref, dg1_ref, h1p_ref, dinv_ref):
    deg = jnp.sum(dg0_ref[0] + dg1_ref[0], axis=1, keepdims=True) * (1.0 / DEGW)
    dinv = lax.rsqrt(deg + 1.0)
    t1 = jnp.dot(x_ref[...], w1_ref[...], preferred_element_type=jnp.float32)
    h1p_ref[...] = t1 * dinv
    dinv_ref[...] = dinv


_prep_call = pl.pallas_call(
    _prep_body,
    grid=(NB,),
    in_specs=[
        pl.BlockSpec((BR, D), lambda i: (i, 0)),
        pl.BlockSpec((D, H), lambda i: (0, 0)),
        pl.BlockSpec((1, BR, DEGW), lambda i: (0, i, 0)),
        pl.BlockSpec((1, BR, DEGW), lambda i: (1, i, 0)),
    ],
    out_specs=(pl.BlockSpec((BR, H), lambda i: (i, 0)),
               pl.BlockSpec((BR, 1), lambda i: (i, 0))),
    out_shape=(jax.ShapeDtypeStruct((N, H), jnp.float32),
               jax.ShapeDtypeStruct((N, 1), jnp.float32)),
)


def _mid_body(ac0_ref, ac1_ref, hp_ref, dinv_ref, w2_ref, b1_ref, out_ref):
    a = (ac0_ref[0] + ac1_ref[0] + hp_ref[...]) * dinv_ref[...] + b1_ref[...]
    h = jnp.maximum(a, 0.0)
    out_ref[...] = jnp.dot(h, w2_ref[...],
                           preferred_element_type=jnp.float32) * dinv_ref[...]


_mid_call = pl.pallas_call(
    _mid_body,
    grid=(NB,),
    in_specs=[
        pl.BlockSpec((1, BR, H), lambda i: (0, i, 0)),
        pl.BlockSpec((1, BR, H), lambda i: (1, i, 0)),
        pl.BlockSpec((BR, H), lambda i: (i, 0)),
        pl.BlockSpec((BR, 1), lambda i: (i, 0)),
        pl.BlockSpec((H, H), lambda i: (0, 0)),
        pl.BlockSpec((H,), lambda i: (0,)),
    ],
    out_specs=pl.BlockSpec((BR, H), lambda i: (i, 0)),
    out_shape=jax.ShapeDtypeStruct((N, H), jnp.float32),
)


def _fin_body(accp_ref, hp_ref, dinv_ref, b2_ref, wfc_ref, bfc_ref, batch_ref,
              out_ref):
    sn = pl.ds(0, N)
    a = ((accp_ref[0, sn] + accp_ref[1, sn] + hp_ref[...])
         * dinv_ref[...] + b2_ref[...])
    h = jnp.maximum(a, 0.0)
    p = jnp.dot(h, wfc_ref[...], preferred_element_type=jnp.float32)
    oh = (batch_ref[...] == lax.broadcasted_iota(jnp.int32, (N, G), 1))
    ohf = oh.astype(jnp.float32)
    hp2 = jnp.concatenate([p, jnp.ones((N, 1), jnp.float32)], axis=1)
    sc2 = lax.dot_general(ohf, hp2, (((0,), (0,)), ((), ())),
                          preferred_element_type=jnp.float32)
    out_ref[...] = sc2[:, 0] / jnp.maximum(sc2[:, 1], 1.0) + bfc_ref[...]


_fin_call = pl.pallas_call(
    _fin_body,
    out_shape=jax.ShapeDtypeStruct((G,), jnp.float32),
)


def kernel(x, edge_index, batch, W1, b1, W2, b2, Wfc, bfc):
    edge3 = edge_index.reshape(2, NCHG, CH)
    batch2 = batch.reshape(N, 1)

    deg_kernel, msg_kernel = _build_sc_kernels()
    degp = deg_kernel(edge3).reshape(NCORES, NPAD, DEGW)
    h1p, dinv = _prep_call(x, W1, degp, degp)
    acc1 = msg_kernel(h1p, edge3).reshape(NCORES, NPAD, H)
    h2p = _mid_call(acc1, acc1, h1p, dinv, W2, b1)
    acc2 = msg_kernel(h2p, edge3).reshape(NCORES, NPAD, H)
    return _fin_call(acc2, h2p, dinv, b2, Wfc, bfc, batch2)


# speedup vs baseline: 1.0246x; 1.0246x over previous
import functools

import jax
import jax.numpy as jnp
from jax import lax
from jax.experimental import pallas as pl
from jax.experimental.pallas import tpu as pltpu
from jax.experimental.pallas import tpu_sc as plsc

N = 10000
E = 320000
D = 128
H = 32
G = 64

NCORES = 2
NSUB = 16
NW = NCORES * NSUB
CH = 128
NCHG = E // CH
NCHW = NCHG // NW
NEXTRA = NCHG - NCHW * NW
NPAD = 10240
RPT = NPAD // NSUB
DEGW = 16


def _deg_body(edge_hbm, out_hbm, dst_v, ones_v, zero_v, deg_sh, sem):
    cid = lax.axis_index("c")
    sid = lax.axis_index("s")
    wid = sid * NCORES + cid
    pltpu.sync_copy(edge_hbm.at[1, pl.ds(wid * NCHW, NCHW)],
                    dst_v.at[pl.ds(0, NCHW)])

    @pl.when(wid < NEXTRA)
    def _load_extra():
        pltpu.sync_copy(edge_hbm.at[1, NCHW * NW + wid], dst_v.at[NCHW])

    def _fill(r, carry):
        ones_v[r, :] = jnp.ones((DEGW,), jnp.float32)
        zero_v[r, :] = jnp.zeros((DEGW,), jnp.float32)
        return carry

    lax.fori_loop(0, CH, _fill, 0)
    for q in range(RPT // CH):
        pltpu.sync_copy(zero_v, deg_sh.at[pl.ds(sid * RPT + q * CH, CH)])
    plsc.subcore_barrier()

    def _scat(k, carry):
        for b in range(6):
            pltpu.async_copy(ones_v, deg_sh.at[dst_v.at[6 * k + b]], sem,
                             add=True)
        for b in range(6):
            pltpu.make_async_copy(ones_v, deg_sh.at[dst_v.at[6 * k + b]],
                                  sem).wait()
        return carry

    lax.fori_loop(0, NCHW // 6, _scat, 0)

    @pl.when(wid < NEXTRA)
    def _scat_extra():
        pltpu.sync_copy(ones_v, deg_sh.at[dst_v.at[NCHW]], add=True)

    plsc.subcore_barrier()
    pltpu.sync_copy(deg_sh.at[pl.ds(sid * RPT, RPT)],
                    out_hbm.at[pl.ds(cid * NPAD + sid * RPT, RPT)])


def _msg_body(h_hbm, edge_hbm, out_hbm,
              src_v, dst_v, buf0, buf1, h_sh, acc_sh, sem0, sem1):
    cid = lax.axis_index("c")
    sid = lax.axis_index("s")
    wid = sid * NCORES + cid
    pltpu.sync_copy(edge_hbm.at[0, pl.ds(wid * NCHW, NCHW)],
                    src_v.at[pl.ds(0, NCHW)])
    pltpu.sync_copy(edge_hbm.at[1, pl.ds(wid * NCHW, NCHW)],
                    dst_v.at[pl.ds(0, NCHW)])

    @pl.when(wid < NEXTRA)
    def _load_extra():
        pltpu.sync_copy(edge_hbm.at[0, NCHW * NW + wid], src_v.at[NCHW])
        pltpu.sync_copy(edge_hbm.at[1, NCHW * NW + wid], dst_v.at[NCHW])

    def _zfill(r, carry):
        buf0[r, pl.ds(0, 16)] = jnp.zeros((16,), jnp.float32)
        buf0[r, pl.ds(16, 16)] = jnp.zeros((16,), jnp.float32)
        return carry

    lax.fori_loop(0, CH, _zfill, 0)
    sl = pl.ds(sid * RPT, RPT)
    for q in range(RPT // CH):
        pltpu.sync_copy(buf0, acc_sh.at[pl.ds(sid * RPT + q * CH, CH)])
    pltpu.sync_copy(h_hbm.at[sl], h_sh.at[sl])
    plsc.subcore_barrier()
    pltpu.async_copy(h_sh.at[src_v.at[0]], buf0, sem0)

    def _pair(k, carry):
        j = 2 * k
        pltpu.make_async_copy(h_sh.at[src_v.at[j]], buf0, sem0).wait()
        pltpu.async_copy(h_sh.at[src_v.at[j + 1]], buf1, sem1)
        pltpu.sync_copy(buf0, acc_sh.at[dst_v.at[j]], add=True)
        pltpu.make_async_copy(h_sh.at[src_v.at[j + 1]], buf1, sem1).wait()

        @pl.when(k < NCHW // 2 - 1)
        def _prefetch():
            pltpu.async_copy(h_sh.at[src_v.at[j + 2]], buf0, sem0)

        pltpu.sync_copy(buf1, acc_sh.at[dst_v.at[j + 1]], add=True)
        return carry

    lax.fori_loop(0, NCHW // 2, _pair, 0)

    @pl.when(wid < NEXTRA)
    def _extra():
        pltpu.async_copy(h_sh.at[src_v.at[NCHW]], buf0, sem0).wait()
        pltpu.sync_copy(buf0, acc_sh.at[dst_v.at[NCHW]], add=True)

    plsc.subcore_barrier()
    pltpu.sync_copy(acc_sh.at[sl],
                    out_hbm.at[pl.ds(cid * NPAD + sid * RPT, RPT)])


@functools.cache
def _build_sc_kernels():
    mesh = plsc.VectorSubcoreMesh(core_axis_name="c", subcore_axis_name="s",
                                  num_cores=NCORES, num_subcores=NSUB)
    params = pltpu.CompilerParams(use_tc_tiling_on_sc=False)
    deg_kernel = pl.kernel(
        _deg_body,
        out_type=jax.ShapeDtypeStruct((NCORES * NPAD, DEGW), jnp.float32),
        mesh=mesh,
        compiler_params=params,
        scratch_types=[
            pltpu.VMEM((NCHW + 1, CH), jnp.int32),
            pltpu.VMEM((CH, DEGW), jnp.float32),
            pltpu.VMEM((CH, DEGW), jnp.float32),
            pltpu.VMEM_SHARED((NPAD, DEGW), jnp.float32),
            pltpu.SemaphoreType.DMA,
        ],
    )
    msg_kernel = pl.kernel(
        _msg_body,
        out_type=jax.ShapeDtypeStruct((NCORES * NPAD, H), jnp.float32),
        mesh=mesh,
        compiler_params=params,
        scratch_types=[
            pltpu.VMEM((NCHW + 1, CH), jnp.int32),
            pltpu.VMEM((NCHW + 1, CH), jnp.int32),
            pltpu.VMEM((CH, H), jnp.float32),
            pltpu.VMEM((CH, H), jnp.float32),
            pltpu.VMEM_SHARED((NPAD, H), jnp.float32),
            pltpu.VMEM_SHARED((NPAD, H), jnp.float32),
            pltpu.SemaphoreType.DMA,
            pltpu.SemaphoreType.DMA,
        ],
    )
    return deg_kernel, msg_kernel


def _prep_body(x_ref, w1_ref, degp_ref, h1p_ref, dinv_ref):
    deg = jnp.sum(degp_ref[0] + degp_ref[1], axis=1, keepdims=True) * (1.0 / DEGW)
    dinv = lax.rsqrt(deg + 1.0)
    t1 = jnp.dot(x_ref[...], w1_ref[...], preferred_element_type=jnp.float32)
    h1p_ref[pl.ds(0, N), :] = t1 * dinv[:N]
    h1p_ref[pl.ds(N, NPAD - N), :] = jnp.zeros((NPAD - N, H), jnp.float32)
    dinv_ref[...] = dinv


_prep_call = pl.pallas_call(
    _prep_body,
    out_shape=(jax.ShapeDtypeStruct((NPAD, H), jnp.float32),
               jax.ShapeDtypeStruct((NPAD, 1), jnp.float32)),
)


def _mid_body(accp_ref, hp_ref, dinv_ref, w2_ref, b1_ref, out_ref):
    a = (accp_ref[0] + accp_ref[1] + hp_ref[...]) * dinv_ref[...] + b1_ref[...]
    h = jnp.maximum(a, 0.0)
    out_ref[...] = jnp.dot(h, w2_ref[...],
                           preferred_element_type=jnp.float32) * dinv_ref[...]


_mid_call = pl.pallas_call(
    _mid_body,
    out_shape=jax.ShapeDtypeStruct((NPAD, H), jnp.float32),
)


def _fin_body(accp_ref, hp_ref, dinv_ref, b2_ref, wfc_ref, bfc_ref, batch_ref,
              out_ref):
    sn = pl.ds(0, N)
    a = ((accp_ref[0, sn] + accp_ref[1, sn] + hp_ref[sn, :])
         * dinv_ref[sn, :] + b2_ref[...])
    h = jnp.maximum(a, 0.0)
    p = jnp.dot(h, wfc_ref[...], preferred_element_type=jnp.float32)
    oh = (batch_ref[...] == lax.broadcasted_iota(jnp.int32, (N, G), 1))
    ohf = oh.astype(jnp.float32)
    hp2 = jnp.concatenate([p, jnp.ones((N, 1), jnp.float32)], axis=1)
    sc2 = lax.dot_general(ohf, hp2, (((0,), (0,)), ((), ())),
                          preferred_element_type=jnp.float32)
    out_ref[...] = sc2[:, 0] / jnp.maximum(sc2[:, 1], 1.0) + bfc_ref[...]


_fin_call = pl.pallas_call(
    _fin_body,
    out_shape=jax.ShapeDtypeStruct((G,), jnp.float32),
)


def kernel(x, edge_index, batch, W1, b1, W2, b2, Wfc, bfc):
    edge3 = edge_index.reshape(2, NCHG, CH)
    batch2 = batch.reshape(N, 1)

    deg_kernel, msg_kernel = _build_sc_kernels()
    degp = deg_kernel(edge3).reshape(NCORES, NPAD, DEGW)
    h1p, dinv = _prep_call(x, W1, degp)
    acc1 = msg_kernel(h1p, edge3).reshape(NCORES, NPAD, H)
    h2p = _mid_call(acc1, h1p, dinv, W2, b1)
    acc2 = msg_kernel(h2p, edge3).reshape(NCORES, NPAD, H)
    return _fin_call(acc2, h2p, dinv, b2, Wfc, bfc, batch2)
